# baseline (speedup 1.0000x reference)
import functools

import jax
import jax.numpy as jnp
from jax.experimental import pallas as pl
from jax.experimental.pallas import tpu as pltpu


def _matmul_bias_kernel(h_ref, w_ref, b_ref, o_ref):
    o_ref[...] = jnp.dot(h_ref[...], w_ref[...],
                         preferred_element_type=jnp.float32) + b_ref[...]


def _matmul_bias(h, w, b):
    n, k = h.shape
    m = w.shape[1]
    blk = 1000
    return pl.pallas_call(
        _matmul_bias_kernel,
        grid=(n // blk,),
        in_specs=[
            pl.BlockSpec((blk, k), lambda i: (i, 0)),
            pl.BlockSpec((k, m), lambda i: (0, 0)),
            pl.BlockSpec((1, m), lambda i: (0, 0)),
        ],
        out_specs=pl.BlockSpec((blk, m), lambda i: (i, 0)),
        out_shape=jax.ShapeDtypeStruct((n, m), jnp.float32),
    )(h, w, b.reshape(1, m))


def kernel(x, edge_index, edge_label, weight, bias, trans_weight,
           attention_weight, edge_attention_weight, agg_weights):
    n = x.shape[0]
    row = edge_index[0]
    col = edge_index[1]

    elt = edge_label @ trans_weight

    xr = x[row]
    xc = x[col]
    node_features = xr * xc
    node_attention = jax.nn.leaky_relu(node_features @ attention_weight, negative_slope=0.01)
    edge_attention = jax.nn.leaky_relu(elt @ edge_attention_weight, negative_slope=0.01)
    attw = jax.nn.sigmoid(node_attention + edge_attention)

    wef = elt * attw
    wnf = xc * attw
    wsrc = xr * attw

    aw = jax.nn.softmax(agg_weights)

    ones = jnp.ones((row.shape[0],), dtype=jnp.float32)
    counts_row = jax.ops.segment_sum(ones, row, num_segments=n)
    counts_col = jax.ops.segment_sum(ones, col, num_segments=n)

    def combine(data, idx, counts):
        s = jax.ops.segment_sum(data, idx, num_segments=n)
        m = jax.ops.segment_max(data, idx, num_segments=n)
        m = jnp.where(counts[:, None] > 0, m, 0.0)
        return (aw[0] / jnp.maximum(counts, 1.0))[:, None] * s + aw[1] * m + aw[2] * s

    opinion = combine(wef, row, counts_row)
    out = combine(wnf, row, counts_row)
    inn_opinion = combine(wef, col, counts_col)
    inn = combine(wsrc, col, counts_col)

    h = jnp.concatenate([out, opinion, inn, inn_opinion], axis=1)
    return _matmul_bias(h, weight, bias)


# speedup vs baseline: 1.1640x; 1.1640x over previous
import functools

import jax
import jax.numpy as jnp
from jax import lax
from jax.experimental import pallas as pl
from jax.experimental.pallas import tpu as pltpu
from jax.experimental.pallas import tpu_sc as plsc


def _matmul_bias_kernel(h_ref, w_ref, b_ref, o_ref):
    o_ref[...] = jnp.dot(h_ref[...], w_ref[...],
                         preferred_element_type=jnp.float32) + b_ref[...]


def _matmul_bias(h, w, b):
    n, k = h.shape
    m = w.shape[1]
    blk = 1000
    return pl.pallas_call(
        _matmul_bias_kernel,
        grid=(n // blk,),
        in_specs=[
            pl.BlockSpec((blk, k), lambda i: (i, 0)),
            pl.BlockSpec((k, m), lambda i: (0, 0)),
            pl.BlockSpec((1, m), lambda i: (0, 0)),
        ],
        out_specs=pl.BlockSpec((blk, m), lambda i: (i, 0)),
        out_shape=jax.ShapeDtypeStruct((n, m), jnp.float32),
    )(h, w, b.reshape(1, m))


def _sc_gather2(x, row, col):
    e = row.shape[0]
    n, d = x.shape
    info = plsc.get_sparse_core_info()
    nw = info.num_cores * info.num_subcores
    per_w = e // nw
    chunk = 400
    nchunks = per_w // chunk
    assert per_w % chunk == 0 and chunk % 8 == 0
    mesh = plsc.VectorSubcoreMesh(core_axis_name="c", subcore_axis_name="s")

    @functools.partial(
        pl.kernel, mesh=mesh,
        out_type=[jax.ShapeDtypeStruct((e, d), jnp.float32),
                  jax.ShapeDtypeStruct((e, d), jnp.float32)],
        scratch_types=[pltpu.VMEM((chunk,), jnp.int32),
                       pltpu.VMEM((chunk,), jnp.int32),
                       pltpu.VMEM((chunk, d), jnp.float32),
                       pltpu.VMEM((chunk, d), jnp.float32),
                       pltpu.SemaphoreType.DMA],
    )
    def k(x_hbm, row_hbm, col_hbm, xr_hbm, xc_hbm, ir_v, ic_v, br_v, bc_v, sem):
        wid = lax.axis_index("s") * info.num_cores + lax.axis_index("c")

        def body(i, carry):
            base = pl.multiple_of(wid * per_w + i * chunk, chunk)
            pltpu.sync_copy(row_hbm.at[pl.ds(base, chunk)], ir_v)
            pltpu.sync_copy(col_hbm.at[pl.ds(base, chunk)], ic_v)
            pltpu.async_copy(x_hbm.at[ir_v], br_v, sem).wait()
            pltpu.async_copy(x_hbm.at[ic_v], bc_v, sem).wait()
            pltpu.sync_copy(br_v, xr_hbm.at[pl.ds(base, chunk)])
            pltpu.sync_copy(bc_v, xc_hbm.at[pl.ds(base, chunk)])
            return carry

        lax.fori_loop(0, nchunks, body, 0)

    return k(x, row, col)


def kernel(x, edge_index, edge_label, weight, bias, trans_weight,
           attention_weight, edge_attention_weight, agg_weights):
    n = x.shape[0]
    row = edge_index[0]
    col = edge_index[1]

    elt = edge_label @ trans_weight

    xr, xc = _sc_gather2(x, row, col)
    node_features = xr * xc
    node_attention = jax.nn.leaky_relu(node_features @ attention_weight, negative_slope=0.01)
    edge_attention = jax.nn.leaky_relu(elt @ edge_attention_weight, negative_slope=0.01)
    attw = jax.nn.sigmoid(node_attention + edge_attention)

    wef = elt * attw
    wnf = xc * attw
    wsrc = xr * attw

    aw = jax.nn.softmax(agg_weights)

    ones = jnp.ones((row.shape[0],), dtype=jnp.float32)
    counts_row = jax.ops.segment_sum(ones, row, num_segments=n)
    counts_col = jax.ops.segment_sum(ones, col, num_segments=n)

    def combine(data, idx, counts):
        s = jax.ops.segment_sum(data, idx, num_segments=n)
        m = jax.ops.segment_max(data, idx, num_segments=n)
        m = jnp.where(counts[:, None] > 0, m, 0.0)
        return (aw[0] / jnp.maximum(counts, 1.0))[:, None] * s + aw[1] * m + aw[2] * s

    opinion = combine(wef, row, counts_row)
    out = combine(wnf, row, counts_row)
    inn_opinion = combine(wef, col, counts_col)
    inn = combine(wsrc, col, counts_col)

    h = jnp.concatenate([out, opinion, inn, inn_opinion], axis=1)
    return _matmul_bias(h, weight, bias)


# speedup vs baseline: 1.6384x; 1.4075x over previous
import functools

import jax
import jax.numpy as jnp
from jax import lax
from jax.experimental import pallas as pl
from jax.experimental.pallas import tpu as pltpu
from jax.experimental.pallas import tpu_sc as plsc


def _matmul_bias_kernel(h_ref, w_ref, b_ref, o_ref):
    o_ref[...] = jnp.dot(h_ref[...], w_ref[...],
                         preferred_element_type=jnp.float32) + b_ref[...]


def _matmul_bias(h, w, b):
    n, k = h.shape
    m = w.shape[1]
    blk = 1000
    return pl.pallas_call(
        _matmul_bias_kernel,
        grid=(n // blk,),
        in_specs=[
            pl.BlockSpec((blk, k), lambda i: (i, 0)),
            pl.BlockSpec((k, m), lambda i: (0, 0)),
            pl.BlockSpec((1, m), lambda i: (0, 0)),
        ],
        out_specs=pl.BlockSpec((blk, m), lambda i: (i, 0)),
        out_shape=jax.ShapeDtypeStruct((n, m), jnp.float32),
    )(h, w, b.reshape(1, m))


def _sc_gather2(x, row, col):
    e = row.shape[0]
    n, d = x.shape
    info = plsc.get_sparse_core_info()
    nw = info.num_cores * info.num_subcores
    per_w = e // nw
    chunk = 400
    nchunks = per_w // chunk
    assert per_w % chunk == 0 and chunk % 8 == 0
    mesh = plsc.VectorSubcoreMesh(core_axis_name="c", subcore_axis_name="s")

    @functools.partial(
        pl.kernel, mesh=mesh,
        out_type=[jax.ShapeDtypeStruct((e, d), jnp.float32),
                  jax.ShapeDtypeStruct((e, d), jnp.float32)],
        scratch_types=[pltpu.VMEM((chunk,), jnp.int32),
                       pltpu.VMEM((chunk,), jnp.int32),
                       pltpu.VMEM((chunk, d), jnp.float32),
                       pltpu.VMEM((chunk, d), jnp.float32),
                       pltpu.SemaphoreType.DMA],
    )
    def k(x_hbm, row_hbm, col_hbm, xr_hbm, xc_hbm, ir_v, ic_v, br_v, bc_v, sem):
        wid = lax.axis_index("s") * info.num_cores + lax.axis_index("c")

        def body(i, carry):
            base = pl.multiple_of(wid * per_w + i * chunk, chunk)
            pltpu.sync_copy(row_hbm.at[pl.ds(base, chunk)], ir_v)
            pltpu.sync_copy(col_hbm.at[pl.ds(base, chunk)], ic_v)
            pltpu.async_copy(x_hbm.at[ir_v], br_v, sem).wait()
            pltpu.async_copy(x_hbm.at[ic_v], bc_v, sem).wait()
            pltpu.sync_copy(br_v, xr_hbm.at[pl.ds(base, chunk)])
            pltpu.sync_copy(bc_v, xc_hbm.at[pl.ds(base, chunk)])
            return carry

        lax.fori_loop(0, nchunks, body, 0)

    return k(x, row, col)


def _sc_segsums(wef, wnf, wsrc, row, col, zeros_nd):
    e, d = wef.shape
    n = zeros_nd.shape[0]
    info = plsc.get_sparse_core_info()
    nc, ns = info.num_cores, info.num_subcores
    nw = nc * ns
    per_w = e // nw
    k = 80
    nchunks = per_w // k
    rows_per_s = n // ns
    assert per_w % k == 0 and n % ns == 0 and rows_per_s % 8 == 0
    mesh = plsc.VectorSubcoreMesh(core_axis_name="c", subcore_axis_name="s")

    @functools.partial(
        pl.kernel, mesh=mesh,
        out_type=[jax.ShapeDtypeStruct((nc, n, d), jnp.float32)] * 4,
        scratch_types=[pltpu.VMEM((k,), jnp.int32),
                       pltpu.VMEM((k, d), jnp.float32),
                       pltpu.VMEM_SHARED((n, d), jnp.float32)],
    )
    def kk(wef_h, wnf_h, wsrc_h, row_h, col_h, znd_h,
           sa_h, sb_h, sc_h, sd_h,
           idx_v, val_v, acc):
        c = lax.axis_index("c")
        s = lax.axis_index("s")
        wid = s * nc + c
        rbase = s * rows_per_s

        def reduce_one(idx_h, val_h, out_h):
            pltpu.sync_copy(znd_h.at[pl.ds(rbase, rows_per_s)],
                            acc.at[pl.ds(rbase, rows_per_s)])
            plsc.subcore_barrier()

            def body(i, carry):
                base = pl.multiple_of(wid * per_w + i * k, k)
                pltpu.sync_copy(idx_h.at[pl.ds(base, k)], idx_v)
                pltpu.sync_copy(val_h.at[pl.ds(base, k)], val_v)
                pltpu.sync_copy(val_v, acc.at[idx_v], add=True)
                return carry

            lax.fori_loop(0, nchunks, body, 0)
            plsc.subcore_barrier()
            pltpu.sync_copy(acc.at[pl.ds(rbase, rows_per_s)],
                            out_h.at[c, pl.ds(rbase, rows_per_s)])
            plsc.subcore_barrier()

        reduce_one(row_h, wef_h, sa_h)
        reduce_one(row_h, wnf_h, sb_h)
        reduce_one(col_h, wef_h, sc_h)
        reduce_one(col_h, wsrc_h, sd_h)

    return kk(wef, wnf, wsrc, row, col, zeros_nd)


def kernel(x, edge_index, edge_label, weight, bias, trans_weight,
           attention_weight, edge_attention_weight, agg_weights):
    n = x.shape[0]
    row = edge_index[0]
    col = edge_index[1]

    elt = edge_label @ trans_weight

    xr, xc = _sc_gather2(x, row, col)
    node_features = xr * xc
    node_attention = jax.nn.leaky_relu(node_features @ attention_weight, negative_slope=0.01)
    edge_attention = jax.nn.leaky_relu(elt @ edge_attention_weight, negative_slope=0.01)
    attw = jax.nn.sigmoid(node_attention + edge_attention)

    wef = elt * attw
    wnf = xc * attw
    wsrc = xr * attw

    aw = jax.nn.softmax(agg_weights)

    d = x.shape[1]
    n_pad = 10240
    zeros_nd = jnp.zeros((n_pad, d), jnp.float32)
    p_wef_row, p_wnf_row, p_wef_col, p_wsrc_col = _sc_segsums(
        wef, wnf, wsrc, row, col, zeros_nd)
    s_wef_row = (p_wef_row[0] + p_wef_row[1])[:n]
    s_wnf_row = (p_wnf_row[0] + p_wnf_row[1])[:n]
    s_wef_col = (p_wef_col[0] + p_wef_col[1])[:n]
    s_wsrc_col = (p_wsrc_col[0] + p_wsrc_col[1])[:n]
    ones = jnp.ones((row.shape[0],), dtype=jnp.float32)
    counts_row = jax.ops.segment_sum(ones, row, num_segments=n)
    counts_col = jax.ops.segment_sum(ones, col, num_segments=n)

    def combine(s, data, idx, counts):
        m = jax.ops.segment_max(data, idx, num_segments=n)
        m = jnp.where(counts[:, None] > 0, m, 0.0)
        return (aw[0] / jnp.maximum(counts, 1.0))[:, None] * s + aw[1] * m + aw[2] * s

    opinion = combine(s_wef_row, wef, row, counts_row)
    out = combine(s_wnf_row, wnf, row, counts_row)
    inn_opinion = combine(s_wef_col, wef, col, counts_col)
    inn = combine(s_wsrc_col, wsrc, col, counts_col)

    h = jnp.concatenate([out, opinion, inn, inn_opinion], axis=1)
    return _matmul_bias(h, weight, bias)
